# baseline (speedup 1.0000x reference)
import jax
import jax.numpy as jnp
from jax import lax
from jax.experimental import pallas as pl
from jax.experimental.pallas import tpu as pltpu
from jax.experimental.pallas import tpu_sc as plsc

_NC = 2
_NS = 16
_NW = _NC * _NS
_L = 16
_C = 128
_NB = 16
_DK = 16
_WBR = 128


def _round_up(a, b):
    return (a + b - 1) // b * b


def _bcast_lane(v16, j):
    idx = jnp.full((_L, 1), j, jnp.int32)
    dn = lax.GatherDimensionNumbers(
        offset_dims=(), collapsed_slice_dims=(0,), start_index_map=(0,))
    return lax.gather(v16, idx, dn, (1,),
                      mode=lax.GatherScatterMode.PROMISE_IN_BOUNDS)


def _sc_mesh():
    return plsc.VectorSubcoreMesh(
        core_axis_name="c", subcore_axis_name="s",
        num_cores=_NC, num_subcores=_NS)


def _make_deg_kernel(epad, npad):
    ew = epad // _NW
    nchunk = ew // _C
    rpt = npad // _NS

    def body(col_hbm, w_hbm, out_hbm, cola, wa, zbuf, acc, sem):
        cid = lax.axis_index("c")
        sid = lax.axis_index("s")
        wid = sid * _NC + cid
        pltpu.sync_copy(col_hbm.at[wid], cola)
        pltpu.sync_copy(w_hbm.at[wid], wa)
        z = jnp.zeros((_L,), jnp.float32)

        def zero_body(i, carry):
            zbuf[pl.ds(i * _L, _L)] = z
            return carry

        lax.fori_loop(0, rpt // _L, zero_body, 0)
        pltpu.sync_copy(zbuf, acc.at[pl.ds(sid * rpt, rpt)])
        plsc.subcore_barrier()

        def group(g, carry):
            descs = []
            for b in range(_DK):
                i = g * _DK + b
                descs.append(pltpu.async_copy(
                    wa.at[i], acc.at[cola.at[i]], sem, add=True))
            for dsc in descs:
                dsc.wait()
            return carry

        lax.fori_loop(0, nchunk // _DK, group, 0)
        plsc.subcore_barrier()
        pltpu.sync_copy(acc.at[pl.ds(sid * rpt, rpt)], zbuf)
        pltpu.sync_copy(zbuf, out_hbm.at[cid, pl.ds(sid * rpt, rpt)])

    return pl.kernel(
        body,
        out_type=jax.ShapeDtypeStruct((_NC, npad), jnp.float32),
        mesh=_sc_mesh(),
        scratch_types=[
            pltpu.VMEM((nchunk, _C), jnp.int32),
            pltpu.VMEM((nchunk, _C), jnp.float32),
            pltpu.VMEM((rpt,), jnp.float32),
            pltpu.VMEM_SHARED((npad,), jnp.float32),
            pltpu.SemaphoreType.DMA,
        ],
    )


def _make_msg_kernel(epad, npad, d):
    ew = epad // _NW
    nchunk = ew // _C
    nblocks = nchunk // _NB
    rpt = npad // _NS
    nwb = rpt // _C

    def body(h_hbm, row_hbm, col_hbm, w_hbm, z_hbm, out_hbm,
             rowa, cola, wa, msgs0, msgs1, acc, sem0, sem1):
        cid = lax.axis_index("c")
        sid = lax.axis_index("s")
        wid = sid * _NC + cid

        pltpu.sync_copy(z_hbm, msgs0)
        for k in range(nwb):
            pltpu.sync_copy(msgs0, acc.at[pl.ds(sid * rpt + k * _C, _C)])
        plsc.subcore_barrier()

        bufs = (msgs0, msgs1)
        sems = (sem0, sem1)
        for bi in range(nblocks):
            pltpu.sync_copy(row_hbm.at[wid, pl.ds(bi * _NB, _NB)], rowa)
            pltpu.sync_copy(col_hbm.at[wid, pl.ds(bi * _NB, _NB)], cola)
            pltpu.sync_copy(w_hbm.at[wid, pl.ds(bi * _NB, _NB)], wa)
            for b in range(2):
                pltpu.async_copy(h_hbm.at[rowa.at[b]], bufs[b], sems[b])

            def pairk(k, carry):
                for b in range(2):
                    i = 2 * k + b
                    buf, sem = bufs[b], sems[b]
                    pltpu.make_async_copy(
                        h_hbm.at[rowa.at[i]], buf, sem).wait()

                    def scale(g, c2):
                        wvv = wa[i, pl.ds(g * _L, _L)]
                        for j in range(_L):
                            s = _bcast_lane(wvv, j)
                            e2 = g * _L + j
                            for dch in range(d // _L):
                                sl = pl.ds(dch * _L, _L)
                                buf[e2, sl] = buf[e2, sl] * s
                        return c2

                    lax.fori_loop(0, _C // _L, scale, 0)
                    pltpu.sync_copy(buf, acc.at[cola.at[i]], add=True)

                    @pl.when(i + 2 < _NB)
                    def _fire():
                        pltpu.async_copy(
                            h_hbm.at[rowa.at[i + 2]], buf, sem)
                return carry

            lax.fori_loop(0, _NB // 2, pairk, 0)

        plsc.subcore_barrier()
        for k in range(nwb):
            off = sid * rpt + k * _C
            pltpu.sync_copy(acc.at[pl.ds(off, _C)], msgs0)
            pltpu.sync_copy(msgs0, out_hbm.at[cid, pl.ds(off, _C)])

    return pl.kernel(
        body,
        out_type=jax.ShapeDtypeStruct((_NC, npad, d), jnp.float32),
        mesh=_sc_mesh(),
        scratch_types=[
            pltpu.VMEM((_NB, _C), jnp.int32),
            pltpu.VMEM((_NB, _C), jnp.int32),
            pltpu.VMEM((_NB, _C), jnp.float32),
            pltpu.VMEM((_C, d), jnp.float32),
            pltpu.VMEM((_C, d), jnp.float32),
            pltpu.VMEM_SHARED((npad, d), jnp.float32),
            pltpu.SemaphoreType.DMA,
            pltpu.SemaphoreType.DMA,
        ],
    )


def _tc1_body(deg_ref, x_ref, w_ref, dis_ref, h_ref):
    n = x_ref.shape[0]
    deg = deg_ref[0] + deg_ref[1]
    dis = jnp.where(deg > 0.0, lax.rsqrt(deg), 0.0)
    dis_ref[...] = dis
    h = jnp.dot(x_ref[...], w_ref[...], precision=lax.Precision.HIGHEST)
    h_ref[...] = h * dis[:n]


def _tc2_body(s_ref, dis_ref, b_ref, w_ref, out_ref):
    n = out_ref.shape[0]
    s = s_ref[0, :n] + s_ref[1, :n]
    dis = dis_ref[:n]
    g = jnp.maximum(s * dis + b_ref[...][None, :], 0.0)
    out_ref[...] = jnp.dot(
        g, w_ref[...], precision=lax.Precision.HIGHEST) * dis


def _tc3_body(s_ref, dis_ref, b_ref, x_ref, out_ref):
    n = x_ref.shape[0]
    s = s_ref[0, :n] + s_ref[1, :n]
    out_ref[...] = (
        jnp.maximum(s * dis_ref[:n] + b_ref[...][None, :], 0.0)
        + x_ref[...])


def kernel(x, adj, edge_weights, W1, b1, W2, b2):
    n, d = x.shape
    e = edge_weights.shape[0]
    npad = _round_up(n, _NS * _C)
    epad = _round_up(e, _NW * _C * _NB)
    nchunk = epad // _NW // _C

    row = adj[0].astype(jnp.int32)
    col = adj[1].astype(jnp.int32)
    w = edge_weights.astype(jnp.float32)
    if epad > e:
        pz = epad - e
        row = jnp.concatenate([row, jnp.zeros((pz,), jnp.int32)])
        col = jnp.concatenate([col, jnp.zeros((pz,), jnp.int32)])
        w = jnp.concatenate([w, jnp.zeros((pz,), jnp.float32)])
    row = row.reshape(_NW, nchunk, _C)
    col = col.reshape(_NW, nchunk, _C)
    w = w.reshape(_NW, nchunk, _C)
    zeros = jnp.zeros((_C, d), jnp.float32)

    f32 = jnp.float32
    deg2 = _make_deg_kernel(epad, npad)(col, w)
    deg2 = deg2.reshape(_NC, npad, 1)

    dis, h1 = pl.pallas_call(
        _tc1_body,
        out_shape=[jax.ShapeDtypeStruct((npad, 1), f32),
                   jax.ShapeDtypeStruct((n, d), f32)],
    )(deg2, x, W1)

    msg = _make_msg_kernel(epad, npad, d)
    s1 = msg(h1, row, col, w, zeros)

    h2 = pl.pallas_call(
        _tc2_body,
        out_shape=jax.ShapeDtypeStruct((n, d), f32),
    )(s1, dis, b1, W2)

    s2 = msg(h2, row, col, w, zeros)

    out = pl.pallas_call(
        _tc3_body,
        out_shape=jax.ShapeDtypeStruct((n, d), f32),
    )(s2, dis, b2, x)

    return out, adj, edge_weights
